# Initial kernel scaffold; baseline (speedup 1.0000x reference)
import functools

import jax
import jax.numpy as jnp
from jax import lax
from jax.experimental import pallas as pl
from jax.experimental.pallas import tpu as pltpu
from jax.experimental.pallas import tpu_sc as plsc

ROW = 160


def _prep_body(h_ref, logits_ref, al_ref, ar_ref,
               el_ref, er_ref, b_ref, pred_ref):
    h = h_ref[...]
    el = jnp.sum(h * al_ref[...], axis=1, keepdims=True)
    er = jnp.sum(h * ar_ref[...], axis=1, keepdims=True)
    gmax = jnp.max(el)
    x = gmax + er
    el_ref[...] = el
    er_ref[...] = er
    b_ref[...] = jnp.where(x >= 0, x, 0.2 * x)
    logits = logits_ref[...]
    ncls = logits.shape[1]
    m = jnp.max(logits, axis=1, keepdims=True)
    iot = lax.broadcasted_iota(jnp.int32, logits.shape, 1)
    pred_ref[...] = jnp.min(jnp.where(logits == m, iot, ncls), axis=1,
                            keepdims=True).astype(jnp.int32)


def _make_edge_call(N, E, D):
    info = plsc.get_sparse_core_info()
    NC, NS, L = info.num_cores, info.num_subcores, info.num_lanes
    NW = NC * NS
    assert E % NW == 0 and N % NS == 0 and D % L == 0
    EPW = E // NW
    CH = 80
    assert EPW % CH == 0
    NCHUNK = EPW // CH
    RPT = N // NS
    NZ, ZREM = RPT // CH, RPT % CH

    mesh = plsc.VectorSubcoreMesh(core_axis_name="c", subcore_axis_name="s")

    @functools.partial(
        pl.kernel,
        out_type=jax.ShapeDtypeStruct((NC, N, ROW), jnp.float32),
        mesh=mesh,
        scratch_types=[
            pltpu.VMEM((N,), jnp.float32),
            pltpu.VMEM((N,), jnp.float32),
            pltpu.VMEM((N,), jnp.float32),
            pltpu.VMEM((N,), jnp.int32),
            pltpu.VMEM((CH,), jnp.int32),
            pltpu.VMEM((CH,), jnp.int32),
            pltpu.VMEM((CH, D), jnp.float32),
            pltpu.VMEM((CH, ROW), jnp.float32),
            pltpu.VMEM((CH,), jnp.float32),
            pltpu.VMEM((CH,), jnp.int32),
            pltpu.VMEM_SHARED((N, ROW), jnp.float32),
            pltpu.SemaphoreType.DMA,
        ],
    )
    def edge_kernel(el_hbm, er_hbm, b_hbm, pred_hbm, src_hbm, dst_hbm, h_hbm,
                    out_hbm, el_t, er_t, b_t, pred_t, src_v, dst_v, hbuf,
                    rowbuf, exbuf, psbuf, acc, sem):
        cid = lax.axis_index("c")
        sid = lax.axis_index("s")
        wid = cid * NS + sid

        pltpu.sync_copy(el_hbm, el_t)
        pltpu.sync_copy(er_hbm, er_t)
        pltpu.sync_copy(b_hbm, b_t)
        pltpu.sync_copy(pred_hbm, pred_t)

        zeros = jnp.zeros((L,), jnp.float32)

        def zrow(i, carry):
            for k in range(ROW // L):
                rowbuf[i, pl.ds(k * L, L)] = zeros
            return carry

        lax.fori_loop(0, CH, zrow, 0)
        base_row = sid * RPT
        for t in range(NZ):
            pltpu.sync_copy(rowbuf, acc.at[pl.ds(base_row + t * CH, CH)])
        if ZREM:
            pltpu.sync_copy(rowbuf.at[pl.ds(0, ZREM)],
                            acc.at[pl.ds(base_row + NZ * CH, ZREM)])
        plsc.subcore_barrier()

        ebase = wid * EPW
        lanes = lax.iota(jnp.int32, L)

        def chunk(ci, carry):
            off = ebase + ci * CH
            pltpu.sync_copy(src_hbm.at[pl.ds(off, CH)], src_v)
            pltpu.sync_copy(dst_hbm.at[pl.ds(off, CH)], dst_v)
            pltpu.async_copy(h_hbm.at[src_v], hbuf, sem).wait()
            for j in range(CH // L):
                s16 = src_v[pl.ds(j * L, L)]
                d16 = dst_v[pl.ds(j * L, L)]
                els = plsc.load_gather(el_t, [s16])
                erd = plsc.load_gather(er_t, [d16])
                bd = plsc.load_gather(b_t, [d16])
                ps = plsc.load_gather(pred_t, [s16])
                e = els + erd
                e = jnp.where(e >= 0, e, jnp.float32(0.2) * e)
                exbuf[pl.ds(j * L, L)] = jnp.exp(e - bd)
                psbuf[pl.ds(j * L, L)] = ps

            def edge(i, carry2):
                ii = jnp.full((L,), i, jnp.int32)
                exs = plsc.load_gather(exbuf, [ii])
                pss = plsc.load_gather(psbuf, [ii])
                m = lanes == pss
                rowbuf[i, pl.ds(D, L)] = jnp.where(m, exs, 0.0)
                rowbuf[i, pl.ds(D + L, L)] = jnp.where(m, 1.0, 0.0)
                for k in range(D // L):
                    rowbuf[i, pl.ds(k * L, L)] = hbuf[i, pl.ds(k * L, L)] * exs
                return carry2

            lax.fori_loop(0, CH, edge, 0)
            pltpu.sync_copy(rowbuf, acc.at[dst_v], add=True)
            return carry

        lax.fori_loop(0, NCHUNK, chunk, 0)
        plsc.subcore_barrier()
        pltpu.sync_copy(acc.at[pl.ds(base_row, RPT)],
                        out_hbm.at[cid, pl.ds(base_row, RPT)])

    return edge_kernel


def _node_body(acc_ref, h_ref, pred_ref, oldz_ref, t1_ref, t2_ref,
               newh_ref, z_ref):
    acc = acc_ref[0] + acc_ref[1]
    D = h_ref.shape[1]
    C = ROW - D - 16
    agg = acc[:, 0:D]
    cnts_raw = acc[:, D:D + C]
    cdeg = acc[:, D + C:ROW]
    deg = jnp.sum(cdeg, axis=1, keepdims=True)
    degs = jnp.maximum(deg, 1.0)
    esum = jnp.sum(cnts_raw, axis=1, keepdims=True)
    present = jnp.sum(cdeg, axis=0, keepdims=True) > 0.0
    inv = 1.0 / (jnp.maximum(esum, 1e-16) * degs)
    cnts = cnts_raw * inv
    pred = pred_ref[...]
    iot = lax.broadcasted_iota(jnp.int32, cnts.shape, 1)
    onehot = (iot == pred).astype(jnp.float32)
    f1 = jnp.sum(cnts * onehot, axis=1, keepdims=True)
    cc = jnp.maximum(cnts, 1e-5)
    f2 = -jnp.sum(jnp.where(present, cc * jnp.log(cc), 0.0), axis=1,
                  keepdims=True)

    def ln(x):
        mu = jnp.mean(x)
        var = jnp.mean((x - mu) ** 2)
        return (x - mu) / jnp.sqrt(var + 1e-5)

    z = (jax.nn.sigmoid(-(ln(f1) - t1_ref[...])) *
         jax.nn.sigmoid(-(ln(f2) - t2_ref[...])))
    gate = jnp.minimum(oldz_ref[...], z)
    normagg = agg * (1.0 / jnp.maximum(esum, 1e-16)) * lax.rsqrt(degs)
    newh_ref[...] = h_ref[...] + gate * normagg
    z_ref[...] = z


def kernel(h, logits, old_z, attn_l, attn_r, tau1, tau2, edge_index):
    N, H, D = h.shape
    E = edge_index.shape[1]
    h2 = h.reshape(N, H * D)
    al = attn_l.reshape(1, H * D)
    ar = attn_r.reshape(1, H * D)

    el, er, b, pred = pl.pallas_call(
        _prep_body,
        out_shape=(
            jax.ShapeDtypeStruct((N, 1), jnp.float32),
            jax.ShapeDtypeStruct((N, 1), jnp.float32),
            jax.ShapeDtypeStruct((N, 1), jnp.float32),
            jax.ShapeDtypeStruct((N, 1), jnp.int32),
        ),
    )(h2, logits, al, ar)

    src = edge_index[0].astype(jnp.int32)
    dst = edge_index[1].astype(jnp.int32)
    edge_call = _make_edge_call(N, E, H * D)
    accs = edge_call(el.reshape(N), er.reshape(N), b.reshape(N),
                     pred.reshape(N), src, dst, h2)

    newh2, z = pl.pallas_call(
        _node_body,
        out_shape=(
            jax.ShapeDtypeStruct((N, H * D), jnp.float32),
            jax.ShapeDtypeStruct((N, 1), jnp.float32),
        ),
    )(accs, h2, pred, old_z.reshape(N, 1), tau1.reshape(1, 1),
      tau2.reshape(1, 1))

    return newh2.reshape(N, H, D), z.reshape(N, H)


# speedup vs baseline: 14.5034x; 14.5034x over previous
import functools

import jax
import jax.numpy as jnp
from jax import lax
from jax.experimental import pallas as pl
from jax.experimental.pallas import tpu as pltpu
from jax.experimental.pallas import tpu_sc as plsc

ROW = 160


def _prep_body(h_ref, logits_ref, al_ref, ar_ref, tab_ref, pred_ref):
    h = h_ref[...]
    el = jnp.sum(h * al_ref[...], axis=1)
    er = jnp.sum(h * ar_ref[...], axis=1)
    gmax = jnp.max(el)
    x = gmax + er
    b = jnp.where(x >= 0, x, 0.2 * x)
    logits = logits_ref[...]
    ncls = logits.shape[1]
    m = jnp.max(logits, axis=1)
    iot = lax.broadcasted_iota(jnp.int32, logits.shape, 1)
    pred = jnp.min(jnp.where(logits == m[:, None], iot, ncls),
                   axis=1).astype(jnp.int32)
    pred_ref[...] = pred
    zero = jnp.zeros_like(el)
    tab_ref[...] = jnp.stack(
        [el, er, b, pred.astype(jnp.float32), zero, zero, zero, zero], axis=0)


def _make_edge_call(N, E, D):
    info = plsc.get_sparse_core_info()
    NC, NS, L = info.num_cores, info.num_subcores, info.num_lanes
    NW = NC * NS
    assert E % NW == 0 and D % L == 0
    EPW = E // NW
    CH = 80
    assert EPW % CH == 0
    NCHUNK = EPW // CH
    NPAD = -(-N // (NS * 8)) * (NS * 8)
    RPT = NPAD // NS
    NZ, ZREM = RPT // CH, RPT % CH
    TW = 8

    mesh = plsc.VectorSubcoreMesh(core_axis_name="c", subcore_axis_name="s")

    @functools.partial(
        pl.kernel,
        out_type=jax.ShapeDtypeStruct((NC, NPAD, ROW), jnp.float32),
        mesh=mesh,
        compiler_params=pltpu.CompilerParams(needs_layout_passes=False,
                                             use_tc_tiling_on_sc=False),
        scratch_types=[
            pltpu.VMEM((CH,), jnp.int32),
            pltpu.VMEM((CH,), jnp.int32),
            pltpu.VMEM((CH, TW), jnp.float32),
            pltpu.VMEM((CH, TW), jnp.float32),
            pltpu.VMEM((CH, D), jnp.float32),
            pltpu.VMEM((CH, ROW), jnp.float32),
            pltpu.VMEM((CH,), jnp.float32),
            pltpu.VMEM((CH,), jnp.float32),
            pltpu.VMEM_SHARED((NPAD, ROW), jnp.float32),
            pltpu.SemaphoreType.DMA,
            pltpu.SemaphoreType.DMA,
            pltpu.SemaphoreType.DMA,
        ],
    )
    def edge_kernel(tab_hbm, src_hbm, dst_hbm, h_hbm,
                    out_hbm, src_v, dst_v, tabs_v, tabd_v, hbuf,
                    rowbuf, exbuf, psbuf, acc, semh, sems, semd):
        cid = lax.axis_index("c")
        sid = lax.axis_index("s")
        wid = cid * NS + sid

        zeros = jnp.zeros((L,), jnp.float32)

        def zrow(i, carry):
            for k in range(ROW // L):
                rowbuf[i, pl.ds(k * L, L)] = zeros
            return carry

        lax.fori_loop(0, CH, zrow, 0)
        base_row = sid * RPT
        for t in range(NZ):
            pltpu.sync_copy(rowbuf, acc.at[pl.ds(base_row + t * CH, CH)])
        if ZREM:
            pltpu.sync_copy(rowbuf.at[pl.ds(0, ZREM)],
                            acc.at[pl.ds(base_row + NZ * CH, ZREM)])
        plsc.subcore_barrier()

        ebase = wid * EPW
        lanes = lax.iota(jnp.int32, L)
        lanes_f = lanes.astype(jnp.float32)

        def chunk(ci, carry):
            off = ebase + ci * CH
            pltpu.sync_copy(src_hbm.at[pl.ds(off, CH)], src_v)
            pltpu.sync_copy(dst_hbm.at[pl.ds(off, CH)], dst_v)
            cph = pltpu.async_copy(h_hbm.at[src_v], hbuf, semh)
            cps = pltpu.async_copy(tab_hbm.at[src_v], tabs_v, sems)
            cpd = pltpu.async_copy(tab_hbm.at[dst_v], tabd_v, semd)
            cps.wait()
            cpd.wait()
            for j in range(CH // L):
                rows = jnp.full((L,), j * L, jnp.int32) + lanes
                els = plsc.load_gather(tabs_v, [rows, jnp.zeros((L,), jnp.int32)])
                ps = plsc.load_gather(tabs_v, [rows, jnp.full((L,), 3, jnp.int32)])
                erd = plsc.load_gather(tabd_v, [rows, jnp.full((L,), 1, jnp.int32)])
                bd = plsc.load_gather(tabd_v, [rows, jnp.full((L,), 2, jnp.int32)])
                e = els + erd
                e = jnp.where(e >= 0, e, jnp.float32(0.2) * e)
                exbuf[pl.ds(j * L, L)] = jnp.exp(e - bd)
                psbuf[pl.ds(j * L, L)] = ps
            cph.wait()

            def edge(i, carry2):
                ii = jnp.full((L,), i, jnp.int32)
                exs = plsc.load_gather(exbuf, [ii])
                pss = plsc.load_gather(psbuf, [ii])
                m = lanes_f == pss
                rowbuf[i, pl.ds(D, L)] = jnp.where(m, exs, 0.0)
                rowbuf[i, pl.ds(D + L, L)] = jnp.where(m, 1.0, 0.0)
                for k in range(D // L):
                    rowbuf[i, pl.ds(k * L, L)] = hbuf[i, pl.ds(k * L, L)] * exs
                return carry2

            lax.fori_loop(0, CH, edge, 0)
            pltpu.sync_copy(rowbuf, acc.at[dst_v], add=True)
            return carry

        lax.fori_loop(0, NCHUNK, chunk, 0)
        plsc.subcore_barrier()
        pltpu.sync_copy(acc.at[pl.ds(base_row, RPT)],
                        out_hbm.at[cid, pl.ds(base_row, RPT)])

    return edge_kernel


def _gate_body(accT_ref, pred_ref, oldz_ref, t1_ref, t2_ref,
               z_ref, coef_ref):
    accT = accT_ref[0] + accT_ref[1]
    C = accT.shape[0] // 2
    cnts_rawT = accT[0:C]
    cdegT = accT[C:2 * C]
    deg = jnp.sum(cdegT, axis=0)
    degs = jnp.maximum(deg, 1.0)
    esum = jnp.sum(cnts_rawT, axis=0)
    present = jnp.sum(cdegT, axis=1, keepdims=True) > 0.0
    inv_esum = jnp.where(esum > 0.0, 1.0 / esum, 0.0)
    cntsT = cnts_rawT * (inv_esum / degs)[None, :]
    pred = pred_ref[...]
    iot = lax.broadcasted_iota(jnp.int32, cntsT.shape, 0)
    onehotT = (iot == pred[None, :]).astype(jnp.float32)
    f1 = jnp.sum(cntsT * onehotT, axis=0)
    cc = jnp.maximum(cntsT, 1e-5)
    f2 = -jnp.sum(jnp.where(present, cc * jnp.log(cc), 0.0), axis=0)

    def ln(x):
        mu = jnp.mean(x)
        var = jnp.mean((x - mu) ** 2)
        return (x - mu) / jnp.sqrt(var + 1e-5)

    t1 = t1_ref[0, 0]
    t2 = t2_ref[0, 0]
    z = jax.nn.sigmoid(-(ln(f1) - t1)) * jax.nn.sigmoid(-(ln(f2) - t2))
    gate = jnp.minimum(oldz_ref[...], z)
    z_ref[...] = z
    coef_ref[...] = gate * inv_esum * lax.rsqrt(degs)


def _update_body(acc_ref, h_ref, coef_ref, newh_ref):
    D = h_ref.shape[1]
    agg = acc_ref[0, :, 0:D] + acc_ref[1, :, 0:D]
    newh_ref[...] = h_ref[...] + coef_ref[...] * agg


def kernel(h, logits, old_z, attn_l, attn_r, tau1, tau2, edge_index):
    N, H, D = h.shape
    E = edge_index.shape[1]
    HD = H * D
    h2 = h.reshape(N, HD)
    al = attn_l.reshape(1, HD)
    ar = attn_r.reshape(1, HD)

    tabT, pred = pl.pallas_call(
        _prep_body,
        out_shape=(
            jax.ShapeDtypeStruct((8, N), jnp.float32),
            jax.ShapeDtypeStruct((N,), jnp.int32),
        ),
    )(h2, logits, al, ar)

    src = edge_index[0].astype(jnp.int32)
    dst = edge_index[1].astype(jnp.int32)
    edge_call = _make_edge_call(N, E, HD)
    accs = edge_call(tabT.T, src, dst, h2)

    acc_smT = jnp.transpose(accs[:, :N, HD:ROW], (0, 2, 1))
    z, coef = pl.pallas_call(
        _gate_body,
        out_shape=(
            jax.ShapeDtypeStruct((N,), jnp.float32),
            jax.ShapeDtypeStruct((N,), jnp.float32),
        ),
    )(acc_smT, pred, old_z.reshape(N), tau1.reshape(1, 1), tau2.reshape(1, 1))

    BB = 2000
    assert N % BB == 0
    newh2 = pl.pallas_call(
        _update_body,
        grid=(N // BB,),
        in_specs=[
            pl.BlockSpec((2, BB, ROW), lambda i: (0, i, 0)),
            pl.BlockSpec((BB, HD), lambda i: (i, 0)),
            pl.BlockSpec((BB, 1), lambda i: (i, 0)),
        ],
        out_specs=pl.BlockSpec((BB, HD), lambda i: (i, 0)),
        out_shape=jax.ShapeDtypeStruct((N, HD), jnp.float32),
    )(accs, h2, coef.reshape(N, 1))

    return newh2.reshape(N, H, D), z.reshape(N, H)


# speedup vs baseline: 21.7794x; 1.5017x over previous
import functools

import jax
import jax.numpy as jnp
from jax import lax
from jax.experimental import pallas as pl
from jax.experimental.pallas import tpu as pltpu
from jax.experimental.pallas import tpu_sc as plsc

ROW = 160


def _prep_body(h_ref, logits_ref, al_ref, ar_ref, tab_ref, pred_ref):
    h = h_ref[...]
    el = jnp.sum(h * al_ref[...], axis=1)
    er = jnp.sum(h * ar_ref[...], axis=1)
    gmax = jnp.max(el)
    x = gmax + er
    b = jnp.where(x >= 0, x, 0.2 * x)
    logits = logits_ref[...]
    ncls = logits.shape[1]
    m = jnp.max(logits, axis=1)
    iot = lax.broadcasted_iota(jnp.int32, logits.shape, 1)
    pred = jnp.min(jnp.where(logits == m[:, None], iot, ncls),
                   axis=1).astype(jnp.int32)
    pred_ref[...] = pred
    zero = jnp.zeros_like(el)
    tab_ref[...] = jnp.stack(
        [el, er, b, pred.astype(jnp.float32), zero, zero, zero, zero], axis=0)


def _make_edge_call(N, E, D):
    info = plsc.get_sparse_core_info()
    NC, NS, L = info.num_cores, info.num_subcores, info.num_lanes
    NW = NC * NS
    assert E % NW == 0 and D % L == 0
    EPW = E // NW
    CH = 40
    assert EPW % CH == 0
    NCHUNK = EPW // CH
    assert NCHUNK >= 8 and (NCHUNK - 2) % 4 >= 0
    NQ = (NCHUNK - 6) // 4
    assert NCHUNK == 2 + 4 * NQ + 4
    NPAD = -(-N // (NS * 8)) * (NS * 8)
    RPT = NPAD // NS
    NZ, ZREM = RPT // CH, RPT % CH
    TW = 8
    UF = 4
    assert CH % (UF * 1) == 0
    L16 = 16
    TPAD = -(-CH // L16) * L16

    mesh = plsc.VectorSubcoreMesh(core_axis_name="c", subcore_axis_name="s")

    @functools.partial(
        pl.kernel,
        out_type=jax.ShapeDtypeStruct((NC, NPAD, ROW), jnp.float32),
        mesh=mesh,
        compiler_params=pltpu.CompilerParams(needs_layout_passes=False,
                                             use_tc_tiling_on_sc=False),
        scratch_types=[
            pltpu.VMEM((4, CH), jnp.int32),
            pltpu.VMEM((4, CH), jnp.int32),
            pltpu.VMEM((2, TPAD, TW), jnp.float32),
            pltpu.VMEM((2, TPAD, TW), jnp.float32),
            pltpu.VMEM((2, CH, D), jnp.float32),
            pltpu.VMEM((2, CH, ROW), jnp.float32),
            pltpu.VMEM((TPAD,), jnp.float32),
            pltpu.VMEM((TPAD,), jnp.float32),
            pltpu.VMEM_SHARED((NPAD, ROW), jnp.float32),
            pltpu.SemaphoreType.DMA,
            pltpu.SemaphoreType.DMA,
            pltpu.SemaphoreType.DMA,
            pltpu.SemaphoreType.DMA,
            pltpu.SemaphoreType.DMA,
            pltpu.SemaphoreType.DMA,
            pltpu.SemaphoreType.DMA,
            pltpu.SemaphoreType.DMA,
            pltpu.SemaphoreType.DMA,
        ],
    )
    def edge_kernel(tab_hbm, src_hbm, dst_hbm, h_hbm,
                    out_hbm, srcb, dstb, tabs, tabd, hbuf,
                    rowbuf, exbuf, psbuf, acc,
                    si0, si1, si2, si3, sg0, sg1, ss0, ss1, sz):
        cid = lax.axis_index("c")
        sid = lax.axis_index("s")
        wid = cid * NS + sid
        semi = (si0, si1, si2, si3)
        semg = (sg0, sg1)
        sems = (ss0, ss1)

        zeros = jnp.zeros((L,), jnp.float32)

        def zrow(i, carry):
            for k in range(ROW // L):
                rowbuf[0, i, pl.ds(k * L, L)] = zeros
            return carry

        lax.fori_loop(0, CH, zrow, 0)
        base_row = sid * RPT
        zcps = []
        for t in range(NZ):
            zcps.append(pltpu.async_copy(
                rowbuf.at[0], acc.at[pl.ds(base_row + t * CH, CH)], sz))
        if ZREM:
            zcps.append(pltpu.async_copy(
                rowbuf.at[0].at[pl.ds(0, ZREM)],
                acc.at[pl.ds(base_row + NZ * CH, ZREM)], sz))
        for cp in zcps:
            cp.wait()
        plsc.subcore_barrier()

        ebase = wid * EPW
        lanes = lax.iota(jnp.int32, L)
        lanes_f = lanes.astype(jnp.float32)
        CHB = CH * 4

        def fire_idx(s, ci):
            off = ebase + ci * CH
            pltpu.async_copy(src_hbm.at[pl.ds(off, CH)], srcb.at[s], semi[s])
            pltpu.async_copy(dst_hbm.at[pl.ds(off, CH)], dstb.at[s], semi[s])

        def wait_idx(s):
            pltpu.make_async_copy(src_hbm.at[pl.ds(0, CH)], srcb.at[s],
                                  semi[s]).wait()
            pltpu.make_async_copy(dst_hbm.at[pl.ds(0, CH)], dstb.at[s],
                                  semi[s]).wait()

        def fire_g(p, s):
            pltpu.async_copy(h_hbm.at[srcb.at[s]], hbuf.at[p], semg[p])
            pltpu.async_copy(tab_hbm.at[srcb.at[s]],
                             tabs.at[p].at[pl.ds(0, CH)], semg[p])
            pltpu.async_copy(tab_hbm.at[dstb.at[s]],
                             tabd.at[p].at[pl.ds(0, CH)], semg[p])

        def wait_g(p):
            pltpu.make_async_copy(h_hbm.at[pl.ds(0, CH)], hbuf.at[p],
                                  semg[p]).wait()
            pltpu.make_async_copy(tab_hbm.at[pl.ds(0, CH)],
                                  tabs.at[p].at[pl.ds(0, CH)], semg[p]).wait()
            pltpu.make_async_copy(tab_hbm.at[pl.ds(0, CH)],
                                  tabd.at[p].at[pl.ds(0, CH)], semg[p]).wait()

        def fire_sc(p, s):
            pltpu.async_copy(rowbuf.at[p], acc.at[dstb.at[s]], sems[p],
                             add=True)

        def wait_sc(p):
            pltpu.make_async_copy(out_hbm.at[0, pl.ds(0, CH)], rowbuf.at[p],
                                  sems[p]).wait()

        def compute(p):
            for j in range(TPAD // L):
                rows = jnp.full((L,), j * L, jnp.int32) + lanes
                els = plsc.load_gather(
                    tabs.at[p], [rows, jnp.zeros((L,), jnp.int32)])
                ps = plsc.load_gather(
                    tabs.at[p], [rows, jnp.full((L,), 3, jnp.int32)])
                erd = plsc.load_gather(
                    tabd.at[p], [rows, jnp.full((L,), 1, jnp.int32)])
                bd = plsc.load_gather(
                    tabd.at[p], [rows, jnp.full((L,), 2, jnp.int32)])
                e = els + erd
                e = jnp.where(e >= 0, e, jnp.float32(0.2) * e)
                exbuf[pl.ds(j * L, L)] = jnp.exp(e - bd)
                psbuf[pl.ds(j * L, L)] = ps

            def edge(g, carry2):
                i0 = g * UF
                for u in range(UF):
                    i = i0 + u
                    ii = jnp.full((L,), i, jnp.int32)
                    exs = plsc.load_gather(exbuf, [ii])
                    pss = plsc.load_gather(psbuf, [ii])
                    m = lanes_f == pss
                    rowbuf[p, i, pl.ds(D, L)] = jnp.where(m, exs, 0.0)
                    rowbuf[p, i, pl.ds(D + L, L)] = jnp.where(m, 1.0, 0.0)
                    for k in range(D // L):
                        rowbuf[p, i, pl.ds(k * L, L)] = (
                            hbuf[p, i, pl.ds(k * L, L)] * exs)
                return carry2

            lax.fori_loop(0, CH // UF, edge, 0)

        def step(ci, s_cur, s_next, s_next2, p, do_wait_sc=True,
                 do_g_next=True, do_idx2=True):
            if do_wait_sc:
                wait_sc(p)
            wait_g(p)
            if do_g_next:
                wait_idx(s_next)
                fire_g(1 - p, s_next)
            if do_idx2:
                fire_idx(s_next2, ci + 2)
            compute(p)
            fire_sc(p, s_cur)

        fire_idx(0, 0)
        fire_idx(1, 1)
        wait_idx(0)
        fire_g(0, 0)
        step(0, 0, 1, 2, 0, do_wait_sc=False)
        step(1, 1, 2, 3, 1, do_wait_sc=False)

        def quad(r, carry):
            ci0 = 2 + 4 * r
            for b in range(4):
                step(ci0 + b, (2 + b) % 4, (3 + b) % 4, b % 4, b % 2)
            return carry

        lax.fori_loop(0, NQ, quad, 0)

        c0 = NCHUNK - 4
        for b in range(4):
            ci = c0 + b
            step(ci, ci % 4, (ci + 1) % 4, (ci + 2) % 4, ci % 2,
                 do_g_next=(ci + 1 < NCHUNK), do_idx2=(ci + 2 < NCHUNK))
        wait_sc(0)
        wait_sc(1)

        plsc.subcore_barrier()
        pltpu.sync_copy(acc.at[pl.ds(base_row, RPT)],
                        out_hbm.at[cid, pl.ds(base_row, RPT)])

    return edge_kernel


def _gate_body(accT_ref, pred_ref, oldz_ref, t1_ref, t2_ref,
               z_ref, coef_ref):
    accT = accT_ref[0] + accT_ref[1]
    C = accT.shape[0] // 2
    cnts_rawT = accT[0:C]
    cdegT = accT[C:2 * C]
    deg = jnp.sum(cdegT, axis=0)
    degs = jnp.maximum(deg, 1.0)
    esum = jnp.sum(cnts_rawT, axis=0)
    present = jnp.sum(cdegT, axis=1, keepdims=True) > 0.0
    inv_esum = jnp.where(esum > 0.0, 1.0 / esum, 0.0)
    cntsT = cnts_rawT * (inv_esum / degs)[None, :]
    pred = pred_ref[...]
    iot = lax.broadcasted_iota(jnp.int32, cntsT.shape, 0)
    onehotT = (iot == pred[None, :]).astype(jnp.float32)
    f1 = jnp.sum(cntsT * onehotT, axis=0)
    cc = jnp.maximum(cntsT, 1e-5)
    f2 = -jnp.sum(jnp.where(present, cc * jnp.log(cc), 0.0), axis=0)

    def ln(x):
        mu = jnp.mean(x)
        var = jnp.mean((x - mu) ** 2)
        return (x - mu) / jnp.sqrt(var + 1e-5)

    t1 = t1_ref[0, 0]
    t2 = t2_ref[0, 0]
    z = jax.nn.sigmoid(-(ln(f1) - t1)) * jax.nn.sigmoid(-(ln(f2) - t2))
    gate = jnp.minimum(oldz_ref[...], z)
    z_ref[...] = z
    coef_ref[...] = gate * inv_esum * lax.rsqrt(degs)


def _update_body(acc_ref, h_ref, coef_ref, newh_ref):
    D = h_ref.shape[1]
    agg = acc_ref[0, :, 0:D] + acc_ref[1, :, 0:D]
    newh_ref[...] = h_ref[...] + coef_ref[...] * agg


def kernel(h, logits, old_z, attn_l, attn_r, tau1, tau2, edge_index):
    N, H, D = h.shape
    E = edge_index.shape[1]
    HD = H * D
    h2 = h.reshape(N, HD)
    al = attn_l.reshape(1, HD)
    ar = attn_r.reshape(1, HD)

    tabT, pred = pl.pallas_call(
        _prep_body,
        out_shape=(
            jax.ShapeDtypeStruct((8, N), jnp.float32),
            jax.ShapeDtypeStruct((N,), jnp.int32),
        ),
    )(h2, logits, al, ar)

    src = edge_index[0].astype(jnp.int32)
    dst = edge_index[1].astype(jnp.int32)
    edge_call = _make_edge_call(N, E, HD)
    accs = edge_call(tabT.T, src, dst, h2)

    acc_smT = jnp.transpose(accs[:, :N, HD:ROW], (0, 2, 1))
    z, coef = pl.pallas_call(
        _gate_body,
        out_shape=(
            jax.ShapeDtypeStruct((N,), jnp.float32),
            jax.ShapeDtypeStruct((N,), jnp.float32),
        ),
    )(acc_smT, pred, old_z.reshape(N), tau1.reshape(1, 1), tau2.reshape(1, 1))

    BB = 2000
    assert N % BB == 0
    newh2 = pl.pallas_call(
        _update_body,
        grid=(N // BB,),
        in_specs=[
            pl.BlockSpec((2, BB, ROW), lambda i: (0, i, 0)),
            pl.BlockSpec((BB, HD), lambda i: (i, 0)),
            pl.BlockSpec((BB, 1), lambda i: (i, 0)),
        ],
        out_specs=pl.BlockSpec((BB, HD), lambda i: (i, 0)),
        out_shape=jax.ShapeDtypeStruct((N, HD), jnp.float32),
    )(accs, h2, coef.reshape(N, 1))

    return newh2.reshape(N, H, D), z.reshape(N, H)


# speedup vs baseline: 35.5024x; 1.6301x over previous
import functools

import jax
import jax.numpy as jnp
from jax import lax
from jax.experimental import pallas as pl
from jax.experimental.pallas import tpu as pltpu
from jax.experimental.pallas import tpu_sc as plsc

ROW = 160


def _prep_body(h_ref, logits_ref, al_ref, ar_ref, tab_ref, pred_ref):
    h = h_ref[...]
    el = jnp.sum(h * al_ref[...], axis=1)
    er = jnp.sum(h * ar_ref[...], axis=1)
    gmax = jnp.max(el)
    x = gmax + er
    b = jnp.where(x >= 0, x, 0.2 * x)
    logits = logits_ref[...]
    ncls = logits.shape[1]
    m = jnp.max(logits, axis=1)
    iot = lax.broadcasted_iota(jnp.int32, logits.shape, 1)
    pred = jnp.min(jnp.where(logits == m[:, None], iot, ncls),
                   axis=1).astype(jnp.int32)
    pred_ref[...] = pred
    zero = jnp.zeros_like(el)
    tab_ref[...] = jnp.stack(
        [el, er, b, pred.astype(jnp.float32), zero, zero, zero, zero], axis=0)


def _make_edge_call(N, E, D):
    info = plsc.get_sparse_core_info()
    NC, NS, L = info.num_cores, info.num_subcores, info.num_lanes
    NW = NC * NS
    assert E % NW == 0 and D % L == 0
    EPW = E // NW
    CH = 40
    assert EPW % CH == 0
    NCHUNK = EPW // CH
    assert NCHUNK >= 8
    NQ = (NCHUNK - 4) // 4
    NTAIL = NCHUNK - 2 - 4 * NQ
    assert 2 <= NTAIL <= 5
    NPAD = -(-N // (NS * 8)) * (NS * 8)
    RPT = NPAD // NS
    NZ, ZREM = RPT // CH, RPT % CH
    TW = 8
    UF = 4
    assert CH % (UF * 1) == 0
    L16 = 16
    TPAD = -(-CH // L16) * L16

    mesh = plsc.VectorSubcoreMesh(core_axis_name="c", subcore_axis_name="s")

    CS = 32

    @functools.partial(
        pl.kernel,
        out_type=(
            jax.ShapeDtypeStruct((NC, NPAD, D), jnp.float32),
            jax.ShapeDtypeStruct((NC, NPAD, CS), jnp.float32),
        ),
        mesh=mesh,
        compiler_params=pltpu.CompilerParams(needs_layout_passes=False,
                                             use_tc_tiling_on_sc=False),
        scratch_types=[
            pltpu.VMEM((4, CH), jnp.int32),
            pltpu.VMEM((4, CH), jnp.int32),
            pltpu.VMEM((2, TPAD, TW), jnp.float32),
            pltpu.VMEM((2, TPAD, TW), jnp.float32),
            pltpu.VMEM((4, CH, D), jnp.float32),
            pltpu.VMEM((4, CH, CS), jnp.float32),
            pltpu.VMEM((TPAD,), jnp.float32),
            pltpu.VMEM((TPAD,), jnp.float32),
            pltpu.VMEM_SHARED((NPAD, D), jnp.float32),
            pltpu.VMEM_SHARED((NPAD, CS), jnp.float32),
            pltpu.SemaphoreType.DMA,
            pltpu.SemaphoreType.DMA,
            pltpu.SemaphoreType.DMA,
            pltpu.SemaphoreType.DMA,
            pltpu.SemaphoreType.DMA,
            pltpu.SemaphoreType.DMA,
            pltpu.SemaphoreType.DMA,
            pltpu.SemaphoreType.DMA,
            pltpu.SemaphoreType.DMA,
        ],
    )
    def edge_kernel(tab_hbm, src_hbm, dst_hbm, h_hbm,
                    out1_hbm, out2_hbm, srcb, dstb, tabs, tabd, hbuf,
                    smallb, exbuf, psbuf, acc1, acc2,
                    si0, si1, si2, si3, sg0, sg1, ss0, ss1, sz):
        cid = lax.axis_index("c")
        sid = lax.axis_index("s")
        wid = cid * NS + sid
        semi = (si0, si1, si2, si3)
        semg = (sg0, sg1)
        sems = (ss0, ss1)

        zeros = jnp.zeros((L,), jnp.float32)

        def zrow(i, carry):
            for k in range(D // L):
                hbuf[0, i, pl.ds(k * L, L)] = zeros
            for k in range(CS // L):
                smallb[0, i, pl.ds(k * L, L)] = zeros
            return carry

        lax.fori_loop(0, CH, zrow, 0)
        base_row = sid * RPT
        zcps = []
        for t in range(NZ):
            zcps.append(pltpu.async_copy(
                hbuf.at[0], acc1.at[pl.ds(base_row + t * CH, CH)], sz))
            zcps.append(pltpu.async_copy(
                smallb.at[0], acc2.at[pl.ds(base_row + t * CH, CH)], sz))
        if ZREM:
            zcps.append(pltpu.async_copy(
                hbuf.at[0].at[pl.ds(0, ZREM)],
                acc1.at[pl.ds(base_row + NZ * CH, ZREM)], sz))
            zcps.append(pltpu.async_copy(
                smallb.at[0].at[pl.ds(0, ZREM)],
                acc2.at[pl.ds(base_row + NZ * CH, ZREM)], sz))
        for cp in zcps:
            cp.wait()
        plsc.subcore_barrier()

        ebase = wid * EPW
        lanes = lax.iota(jnp.int32, L)
        lanes_f = lanes.astype(jnp.float32)
        CHB = CH * 4

        def fire_idx(s, ci):
            off = ebase + ci * CH
            pltpu.async_copy(src_hbm.at[pl.ds(off, CH)], srcb.at[s], semi[s])
            pltpu.async_copy(dst_hbm.at[pl.ds(off, CH)], dstb.at[s], semi[s])

        def wait_idx(s):
            pltpu.make_async_copy(src_hbm.at[pl.ds(0, CH)], srcb.at[s],
                                  semi[s]).wait()
            pltpu.make_async_copy(dst_hbm.at[pl.ds(0, CH)], dstb.at[s],
                                  semi[s]).wait()

        def fire_g(p, s):
            pltpu.async_copy(h_hbm.at[srcb.at[s]], hbuf.at[s], semg[p])
            pltpu.async_copy(tab_hbm.at[srcb.at[s]],
                             tabs.at[p].at[pl.ds(0, CH)], semg[p])
            pltpu.async_copy(tab_hbm.at[dstb.at[s]],
                             tabd.at[p].at[pl.ds(0, CH)], semg[p])

        def wait_g(p, s):
            pltpu.make_async_copy(h_hbm.at[pl.ds(0, CH)], hbuf.at[s],
                                  semg[p]).wait()
            pltpu.make_async_copy(tab_hbm.at[pl.ds(0, CH)],
                                  tabs.at[p].at[pl.ds(0, CH)], semg[p]).wait()
            pltpu.make_async_copy(tab_hbm.at[pl.ds(0, CH)],
                                  tabd.at[p].at[pl.ds(0, CH)], semg[p]).wait()

        def fire_sc(p, s):
            pltpu.async_copy(hbuf.at[s], acc1.at[dstb.at[s]], sems[p],
                             add=True)
            pltpu.async_copy(smallb.at[s], acc2.at[dstb.at[s]], sems[p],
                             add=True)

        def wait_sc(p, s):
            pltpu.make_async_copy(out1_hbm.at[0, pl.ds(0, CH)], hbuf.at[s],
                                  sems[p]).wait()
            pltpu.make_async_copy(out2_hbm.at[0, pl.ds(0, CH)], smallb.at[s],
                                  sems[p]).wait()

        def compute(p, s):
            for j in range(TPAD // L):
                rows = jnp.full((L,), j * L, jnp.int32) + lanes
                els = plsc.load_gather(
                    tabs.at[p], [rows, jnp.zeros((L,), jnp.int32)])
                ps = plsc.load_gather(
                    tabs.at[p], [rows, jnp.full((L,), 3, jnp.int32)])
                erd = plsc.load_gather(
                    tabd.at[p], [rows, jnp.full((L,), 1, jnp.int32)])
                bd = plsc.load_gather(
                    tabd.at[p], [rows, jnp.full((L,), 2, jnp.int32)])
                e = els + erd
                e = jnp.where(e >= 0, e, jnp.float32(0.2) * e)
                exbuf[pl.ds(j * L, L)] = jnp.exp(e - bd)
                psbuf[pl.ds(j * L, L)] = ps

            def edge(g, carry2):
                i0 = g * UF
                for u in range(UF):
                    i = i0 + u
                    ii = jnp.full((L,), i, jnp.int32)
                    exs = plsc.load_gather(exbuf, [ii])
                    pss = plsc.load_gather(psbuf, [ii])
                    m = lanes_f == pss
                    smallb[s, i, pl.ds(0, L)] = jnp.where(m, exs, 0.0)
                    smallb[s, i, pl.ds(L, L)] = jnp.where(m, 1.0, 0.0)
                    for k in range(D // L):
                        hbuf[s, i, pl.ds(k * L, L)] = (
                            hbuf[s, i, pl.ds(k * L, L)] * exs)
                return carry2

            lax.fori_loop(0, CH // UF, edge, 0)

        def step(ci, s_cur, s_next, s_next2, p, do_wait_sc=True,
                 do_g_next=True, do_idx2=True):
            if do_wait_sc:
                wait_sc(p, s_next2)
            wait_g(p, s_cur)
            if do_g_next:
                wait_idx(s_next)
                fire_g(1 - p, s_next)
            if do_idx2:
                fire_idx(s_next2, ci + 2)
            compute(p, s_cur)
            fire_sc(p, s_cur)

        fire_idx(0, 0)
        fire_idx(1, 1)
        wait_idx(0)
        fire_g(0, 0)
        step(0, 0, 1, 2, 0, do_wait_sc=False)
        step(1, 1, 2, 3, 1, do_wait_sc=False)

        def quad(r, carry):
            ci0 = 2 + 4 * r
            for b in range(4):
                step(ci0 + b, (2 + b) % 4, (3 + b) % 4, b % 4, b % 2)
            return carry

        lax.fori_loop(0, NQ, quad, 0)

        c0 = 2 + 4 * NQ
        for b in range(NTAIL):
            ci = c0 + b
            step(ci, ci % 4, (ci + 1) % 4, (ci + 2) % 4, ci % 2,
                 do_g_next=(ci + 1 < NCHUNK), do_idx2=(ci + 2 < NCHUNK))
        wait_sc(0, (NCHUNK - 2) % 4)
        wait_sc(1, (NCHUNK - 1) % 4)

        plsc.subcore_barrier()
        pltpu.sync_copy(acc1.at[pl.ds(base_row, RPT)],
                        out1_hbm.at[cid, pl.ds(base_row, RPT)])
        pltpu.sync_copy(acc2.at[pl.ds(base_row, RPT)],
                        out2_hbm.at[cid, pl.ds(base_row, RPT)])

    return edge_kernel


def _gate_body(accT_ref, pred_ref, oldz_ref, t1_ref, t2_ref,
               z_ref, coef_ref):
    accT = accT_ref[0] + accT_ref[1]
    C = accT.shape[0] // 2
    cnts_rawT = accT[0:C]
    cdegT = accT[C:2 * C]
    deg = jnp.sum(cdegT, axis=0)
    degs = jnp.maximum(deg, 1.0)
    esum = jnp.sum(cnts_rawT, axis=0)
    present = jnp.sum(cdegT, axis=1, keepdims=True) > 0.0
    inv_esum = jnp.where(esum > 0.0, 1.0 / esum, 0.0)
    cntsT = cnts_rawT * (inv_esum / degs)[None, :]
    pred = pred_ref[...]
    iot = lax.broadcasted_iota(jnp.int32, cntsT.shape, 0)
    onehotT = (iot == pred[None, :]).astype(jnp.float32)
    f1 = jnp.sum(cntsT * onehotT, axis=0)
    cc = jnp.maximum(cntsT, 1e-5)
    f2 = -jnp.sum(jnp.where(present, cc * jnp.log(cc), 0.0), axis=0)

    def ln(x):
        mu = jnp.mean(x)
        var = jnp.mean((x - mu) ** 2)
        return (x - mu) / jnp.sqrt(var + 1e-5)

    t1 = t1_ref[0, 0]
    t2 = t2_ref[0, 0]
    z = jax.nn.sigmoid(-(ln(f1) - t1)) * jax.nn.sigmoid(-(ln(f2) - t2))
    gate = jnp.minimum(oldz_ref[...], z)
    z_ref[...] = z
    coef_ref[...] = gate * inv_esum * lax.rsqrt(degs)


def _update_body(acc_ref, h_ref, coef_ref, newh_ref):
    agg = acc_ref[0] + acc_ref[1]
    newh_ref[...] = h_ref[...] + coef_ref[...] * agg


def kernel(h, logits, old_z, attn_l, attn_r, tau1, tau2, edge_index):
    N, H, D = h.shape
    E = edge_index.shape[1]
    HD = H * D
    h2 = h.reshape(N, HD)
    al = attn_l.reshape(1, HD)
    ar = attn_r.reshape(1, HD)

    tabT, pred = pl.pallas_call(
        _prep_body,
        out_shape=(
            jax.ShapeDtypeStruct((8, N), jnp.float32),
            jax.ShapeDtypeStruct((N,), jnp.int32),
        ),
    )(h2, logits, al, ar)

    src = edge_index[0].astype(jnp.int32)
    dst = edge_index[1].astype(jnp.int32)
    edge_call = _make_edge_call(N, E, HD)
    agg_acc, cnt_acc = edge_call(tabT.T, src, dst, h2)

    acc_smT = jnp.transpose(cnt_acc[:, :N, :], (0, 2, 1))
    z, coef = pl.pallas_call(
        _gate_body,
        out_shape=(
            jax.ShapeDtypeStruct((N,), jnp.float32),
            jax.ShapeDtypeStruct((N,), jnp.float32),
        ),
    )(acc_smT, pred, old_z.reshape(N), tau1.reshape(1, 1), tau2.reshape(1, 1))

    BB = 2000
    assert N % BB == 0
    newh2 = pl.pallas_call(
        _update_body,
        grid=(N // BB,),
        in_specs=[
            pl.BlockSpec((2, BB, HD), lambda i: (0, i, 0)),
            pl.BlockSpec((BB, HD), lambda i: (i, 0)),
            pl.BlockSpec((BB, 1), lambda i: (i, 0)),
        ],
        out_specs=pl.BlockSpec((BB, HD), lambda i: (i, 0)),
        out_shape=jax.ShapeDtypeStruct((N, HD), jnp.float32),
    )(agg_acc, h2, coef.reshape(N, 1))

    return newh2.reshape(N, H, D), z.reshape(N, H)


# speedup vs baseline: 41.0601x; 1.1565x over previous
import functools

import jax
import jax.numpy as jnp
from jax import lax
from jax.experimental import pallas as pl
from jax.experimental.pallas import tpu as pltpu
from jax.experimental.pallas import tpu_sc as plsc

ROW = 160


def _prep_body(h_ref, logits_ref, al_ref, ar_ref, tab_ref, pred_ref):
    h = h_ref[...]
    el = jnp.sum(h * al_ref[...], axis=1)
    er = jnp.sum(h * ar_ref[...], axis=1)
    gmax = jnp.max(el)
    x = gmax + er
    b = jnp.where(x >= 0, x, 0.2 * x)
    logits = logits_ref[...]
    ncls = logits.shape[1]
    m = jnp.max(logits, axis=1)
    iot = lax.broadcasted_iota(jnp.int32, logits.shape, 1)
    pred = jnp.min(jnp.where(logits == m[:, None], iot, ncls),
                   axis=1).astype(jnp.int32)
    pred_ref[...] = pred
    zero = jnp.zeros_like(el)
    tab_ref[...] = jnp.stack(
        [el, er, b, pred.astype(jnp.float32), zero, zero, zero, zero], axis=0)


def _make_edge_call(N, E, D):
    info = plsc.get_sparse_core_info()
    NC, NS, L = info.num_cores, info.num_subcores, info.num_lanes
    NW = NC * NS
    assert E % NW == 0 and D % L == 0
    EPW = E // NW
    CH = 40
    assert EPW % CH == 0
    NCHUNK = EPW // CH
    assert NCHUNK >= 16
    NO = (NCHUNK - 10) // 8
    NTAIL = NCHUNK - 2 - 8 * NO
    assert 8 <= NTAIL <= 15
    NPAD = -(-N // (NS * 8)) * (NS * 8)
    RPT = NPAD // NS
    NZ, ZREM = RPT // CH, RPT % CH
    TW = 8
    UF = 4
    assert CH % (UF * 1) == 0
    L16 = 16
    TPAD = -(-CH // L16) * L16

    mesh = plsc.VectorSubcoreMesh(core_axis_name="c", subcore_axis_name="s")

    CS = 32

    @functools.partial(
        pl.kernel,
        out_type=(
            jax.ShapeDtypeStruct((NC, NPAD, D), jnp.float32),
            jax.ShapeDtypeStruct((NC, NPAD, CS), jnp.float32),
        ),
        mesh=mesh,
        compiler_params=pltpu.CompilerParams(needs_layout_passes=False,
                                             use_tc_tiling_on_sc=False),
        scratch_types=[
            pltpu.VMEM((8, CH), jnp.int32),
            pltpu.VMEM((8, CH), jnp.int32),
            pltpu.VMEM((4, TPAD, TW), jnp.float32),
            pltpu.VMEM((4, TPAD, TW), jnp.float32),
            pltpu.VMEM((4, CH, D), jnp.float32),
            pltpu.VMEM((4, CH, CS), jnp.float32),
            pltpu.VMEM((TPAD,), jnp.float32),
            pltpu.VMEM((TPAD,), jnp.float32),
            pltpu.VMEM_SHARED((NPAD, D), jnp.float32),
            pltpu.VMEM_SHARED((NPAD, CS), jnp.float32),
            pltpu.SemaphoreType.DMA,
            pltpu.SemaphoreType.DMA,
            pltpu.SemaphoreType.DMA,
            pltpu.SemaphoreType.DMA,
            pltpu.SemaphoreType.DMA,
            pltpu.SemaphoreType.DMA,
            pltpu.SemaphoreType.DMA,
            pltpu.SemaphoreType.DMA,
            pltpu.SemaphoreType.DMA,
            pltpu.SemaphoreType.DMA,
            pltpu.SemaphoreType.DMA,
            pltpu.SemaphoreType.DMA,
            pltpu.SemaphoreType.DMA,
            pltpu.SemaphoreType.DMA,
            pltpu.SemaphoreType.DMA,
        ],
    )
    def edge_kernel(tab_hbm, src_hbm, dst_hbm, h_hbm,
                    out1_hbm, out2_hbm, srcb, dstb, tabs, tabd, hbuf,
                    smallb, exbuf, psbuf, acc1, acc2,
                    si0, si1, si2, si3, si4, si5, si6, si7,
                    sg0, sg1, sg2, sg3, ss0, ss1, sz):
        cid = lax.axis_index("c")
        sid = lax.axis_index("s")
        wid = cid * NS + sid
        semi = (si0, si1, si2, si3, si4, si5, si6, si7)
        semg = (sg0, sg1, sg2, sg3)
        sems = (ss0, ss1)

        zeros = jnp.zeros((L,), jnp.float32)

        def zrow(i, carry):
            for k in range(D // L):
                hbuf[0, i, pl.ds(k * L, L)] = zeros
            for k in range(CS // L):
                smallb[0, i, pl.ds(k * L, L)] = zeros
            return carry

        lax.fori_loop(0, CH, zrow, 0)
        base_row = sid * RPT
        zcps = []
        for t in range(NZ):
            zcps.append(pltpu.async_copy(
                hbuf.at[0], acc1.at[pl.ds(base_row + t * CH, CH)], sz))
            zcps.append(pltpu.async_copy(
                smallb.at[0], acc2.at[pl.ds(base_row + t * CH, CH)], sz))
        if ZREM:
            zcps.append(pltpu.async_copy(
                hbuf.at[0].at[pl.ds(0, ZREM)],
                acc1.at[pl.ds(base_row + NZ * CH, ZREM)], sz))
            zcps.append(pltpu.async_copy(
                smallb.at[0].at[pl.ds(0, ZREM)],
                acc2.at[pl.ds(base_row + NZ * CH, ZREM)], sz))
        for cp in zcps:
            cp.wait()
        plsc.subcore_barrier()

        ebase = wid * EPW
        lanes = lax.iota(jnp.int32, L)
        lanes_f = lanes.astype(jnp.float32)
        CHB = CH * 4

        def fire_idx(s, ci):
            off = ebase + ci * CH
            pltpu.async_copy(src_hbm.at[pl.ds(off, CH)], srcb.at[s], semi[s])
            pltpu.async_copy(dst_hbm.at[pl.ds(off, CH)], dstb.at[s], semi[s])

        def wait_idx(s):
            pltpu.make_async_copy(src_hbm.at[pl.ds(0, CH)], srcb.at[s],
                                  semi[s]).wait()
            pltpu.make_async_copy(dst_hbm.at[pl.ds(0, CH)], dstb.at[s],
                                  semi[s]).wait()

        def fire_g(s4, s8):
            pltpu.async_copy(h_hbm.at[srcb.at[s8]], hbuf.at[s4], semg[s4])
            pltpu.async_copy(tab_hbm.at[srcb.at[s8]],
                             tabs.at[s4].at[pl.ds(0, CH)], semg[s4])
            pltpu.async_copy(tab_hbm.at[dstb.at[s8]],
                             tabd.at[s4].at[pl.ds(0, CH)], semg[s4])

        def wait_g(s4):
            pltpu.make_async_copy(h_hbm.at[pl.ds(0, CH)], hbuf.at[s4],
                                  semg[s4]).wait()
            pltpu.make_async_copy(tab_hbm.at[pl.ds(0, CH)],
                                  tabs.at[s4].at[pl.ds(0, CH)],
                                  semg[s4]).wait()
            pltpu.make_async_copy(tab_hbm.at[pl.ds(0, CH)],
                                  tabd.at[s4].at[pl.ds(0, CH)],
                                  semg[s4]).wait()

        def fire_sc(p, s4, s8):
            pltpu.async_copy(hbuf.at[s4], acc1.at[dstb.at[s8]], sems[p],
                             add=True)
            pltpu.async_copy(smallb.at[s4], acc2.at[dstb.at[s8]], sems[p],
                             add=True)

        def wait_sc(p, s4):
            pltpu.make_async_copy(out1_hbm.at[0, pl.ds(0, CH)], hbuf.at[s4],
                                  sems[p]).wait()
            pltpu.make_async_copy(out2_hbm.at[0, pl.ds(0, CH)], smallb.at[s4],
                                  sems[p]).wait()

        def compute(s4):
            p = s4
            for j in range(TPAD // L):
                rows = jnp.full((L,), j * L, jnp.int32) + lanes
                els = plsc.load_gather(
                    tabs.at[p], [rows, jnp.zeros((L,), jnp.int32)])
                ps = plsc.load_gather(
                    tabs.at[p], [rows, jnp.full((L,), 3, jnp.int32)])
                erd = plsc.load_gather(
                    tabd.at[p], [rows, jnp.full((L,), 1, jnp.int32)])
                bd = plsc.load_gather(
                    tabd.at[p], [rows, jnp.full((L,), 2, jnp.int32)])
                e = els + erd
                e = jnp.where(e >= 0, e, jnp.float32(0.2) * e)
                exbuf[pl.ds(j * L, L)] = jnp.exp(e - bd)
                psbuf[pl.ds(j * L, L)] = ps

            def edge(g, carry2):
                i0 = g * UF
                for u in range(UF):
                    i = i0 + u
                    ii = jnp.full((L,), i, jnp.int32)
                    exs = plsc.load_gather(exbuf, [ii])
                    pss = plsc.load_gather(psbuf, [ii])
                    m = lanes_f == pss
                    smallb[s4, i, pl.ds(0, L)] = jnp.where(m, exs, 0.0)
                    smallb[s4, i, pl.ds(L, L)] = jnp.where(m, 1.0, 0.0)
                    for k in range(D // L):
                        hbuf[s4, i, pl.ds(k * L, L)] = (
                            hbuf[s4, i, pl.ds(k * L, L)] * exs)
                return carry2

            lax.fori_loop(0, CH // UF, edge, 0)

        def step(ci, real_ci=None, in_main=False, do_wait_sc=True):
            real = ci if real_ci is None else real_ci
            s4, s8, p = ci % 4, ci % 8, ci % 2
            if do_wait_sc:
                wait_sc(p, (ci + 2) % 4)
            wait_g(s4)
            if in_main or ci + 4 < NCHUNK:
                fire_idx((ci + 4) % 8, real + 4)
            if in_main or ci + 2 < NCHUNK:
                wait_idx((ci + 2) % 8)
                fire_g((ci + 2) % 4, (ci + 2) % 8)
            compute(s4)
            fire_sc(p, s4, s8)

        for c in range(4):
            fire_idx(c, c)
        wait_idx(0)
        fire_g(0, 0)
        wait_idx(1)
        fire_g(1, 1)
        step(0, do_wait_sc=False)
        step(1, do_wait_sc=False)

        def octave(r, carry):
            base = 2 + 8 * r
            for b in range(8):
                step(2 + b, real_ci=base + b, in_main=True)
            return carry

        lax.fori_loop(0, NO, octave, 0)

        c0 = 2 + 8 * NO
        for b in range(NTAIL):
            step(c0 + b)
        wait_sc(0, (NCHUNK - 2) % 4)
        wait_sc(1, (NCHUNK - 1) % 4)

        plsc.subcore_barrier()
        pltpu.sync_copy(acc1.at[pl.ds(base_row, RPT)],
                        out1_hbm.at[cid, pl.ds(base_row, RPT)])
        pltpu.sync_copy(acc2.at[pl.ds(base_row, RPT)],
                        out2_hbm.at[cid, pl.ds(base_row, RPT)])

    return edge_kernel


def _gate_body(accT_ref, pred_ref, oldz_ref, t1_ref, t2_ref,
               z_ref, coef_ref):
    accT = accT_ref[0] + accT_ref[1]
    C = accT.shape[0] // 2
    cnts_rawT = accT[0:C]
    cdegT = accT[C:2 * C]
    deg = jnp.sum(cdegT, axis=0)
    degs = jnp.maximum(deg, 1.0)
    esum = jnp.sum(cnts_rawT, axis=0)
    present = jnp.sum(cdegT, axis=1, keepdims=True) > 0.0
    inv_esum = jnp.where(esum > 0.0, 1.0 / esum, 0.0)
    cntsT = cnts_rawT * (inv_esum / degs)[None, :]
    pred = pred_ref[...]
    iot = lax.broadcasted_iota(jnp.int32, cntsT.shape, 0)
    onehotT = (iot == pred[None, :]).astype(jnp.float32)
    f1 = jnp.sum(cntsT * onehotT, axis=0)
    cc = jnp.maximum(cntsT, 1e-5)
    f2 = -jnp.sum(jnp.where(present, cc * jnp.log(cc), 0.0), axis=0)

    def ln(x):
        mu = jnp.mean(x)
        var = jnp.mean((x - mu) ** 2)
        return (x - mu) / jnp.sqrt(var + 1e-5)

    t1 = t1_ref[0, 0]
    t2 = t2_ref[0, 0]
    z = jax.nn.sigmoid(-(ln(f1) - t1)) * jax.nn.sigmoid(-(ln(f2) - t2))
    gate = jnp.minimum(oldz_ref[...], z)
    z_ref[...] = z
    coef_ref[...] = gate * inv_esum * lax.rsqrt(degs)


def _update_body(acc_ref, h_ref, coef_ref, newh_ref):
    agg = acc_ref[0] + acc_ref[1]
    newh_ref[...] = h_ref[...] + coef_ref[...] * agg


def kernel(h, logits, old_z, attn_l, attn_r, tau1, tau2, edge_index):
    N, H, D = h.shape
    E = edge_index.shape[1]
    HD = H * D
    h2 = h.reshape(N, HD)
    al = attn_l.reshape(1, HD)
    ar = attn_r.reshape(1, HD)

    tabT, pred = pl.pallas_call(
        _prep_body,
        out_shape=(
            jax.ShapeDtypeStruct((8, N), jnp.float32),
            jax.ShapeDtypeStruct((N,), jnp.int32),
        ),
    )(h2, logits, al, ar)

    src = edge_index[0].astype(jnp.int32)
    dst = edge_index[1].astype(jnp.int32)
    edge_call = _make_edge_call(N, E, HD)
    agg_acc, cnt_acc = edge_call(tabT.T, src, dst, h2)

    acc_smT = jnp.transpose(cnt_acc[:, :N, :], (0, 2, 1))
    z, coef = pl.pallas_call(
        _gate_body,
        out_shape=(
            jax.ShapeDtypeStruct((N,), jnp.float32),
            jax.ShapeDtypeStruct((N,), jnp.float32),
        ),
    )(acc_smT, pred, old_z.reshape(N), tau1.reshape(1, 1), tau2.reshape(1, 1))

    BB = 2000
    assert N % BB == 0
    newh2 = pl.pallas_call(
        _update_body,
        grid=(N // BB,),
        in_specs=[
            pl.BlockSpec((2, BB, HD), lambda i: (0, i, 0)),
            pl.BlockSpec((BB, HD), lambda i: (i, 0)),
            pl.BlockSpec((BB, 1), lambda i: (i, 0)),
        ],
        out_specs=pl.BlockSpec((BB, HD), lambda i: (i, 0)),
        out_shape=jax.ShapeDtypeStruct((N, HD), jnp.float32),
    )(agg_acc, h2, coef.reshape(N, 1))

    return newh2.reshape(N, H, D), z.reshape(N, H)
